# baseline (speedup 1.0000x reference)
import functools

import jax
import jax.numpy as jnp
import numpy as np
from jax import lax
from jax.experimental import pallas as pl
from jax.experimental.pallas import tpu as pltpu
from jax.experimental.pallas import tpu_sc as plsc

B = 4
N = 8192
M = 1024
C = 128
K = 32
QB = 128


def _lex_lt(av, ai, bv, bi):
    return (av < bv) | ((av == bv) & (ai < bi))


def _ce_dim0(v, i, s, L):
    out_v = []
    out_i = []
    for g in range(K // (2 * s)):
        l0 = g * 2 * s
        asc = True if L is None else ((l0 // L) % 2 == 0)
        a_v, a_i = v[l0:l0 + s], i[l0:l0 + s]
        b_v, b_i = v[l0 + s:l0 + 2 * s], i[l0 + s:l0 + 2 * s]
        lt = _lex_lt(a_v, a_i, b_v, b_i)
        if not asc:
            lt = ~lt
        out_v += [jnp.where(lt, a_v, b_v), jnp.where(lt, b_v, a_v)]
        out_i += [jnp.where(lt, a_i, b_i), jnp.where(lt, b_i, a_i)]
    if len(out_v) == 1:
        pass
    v = jnp.concatenate(out_v, 0)
    i = jnp.concatenate(out_i, 0)
    return v, i


def _sort_k_dim0(v, i):
    L = 2
    while L <= K:
        s = L // 2
        while s >= 1:
            v, i = _ce_dim0(v, i, s, None if L == K else L)
            s //= 2
        L *= 2
    return v, i


def _merge_round(v, i):
    t2 = v.shape[1] // 2
    av, ai = v[:, :t2], i[:, :t2]
    rev = list(reversed(range(K)))
    bv = jnp.concatenate([v[j:j + 1, t2:] for j in rev], 0)
    bi = jnp.concatenate([i[j:j + 1, t2:] for j in rev], 0)
    lt = _lex_lt(av, ai, bv, bi)
    nv = jnp.where(lt, av, bv)
    ni = jnp.where(lt, ai, bi)
    s = K // 2
    while s >= 1:
        nv, ni = _ce_dim0(nv, ni, s, None)
        s //= 2
    return nv, ni


def _topk_body(p_ref, q_ref, xx_ref, idx_ref):
    b = pl.program_id(0)
    p = p_ref[0]
    q = q_ref[0]
    xx = xx_ref[0]
    e = lax.dot_general(p, q, (((1,), (0,)), ((), ())),
                        preferred_element_type=jnp.float32)
    d = (xx + q[3:4, :]) - 2.0 * e
    d = jnp.maximum(d, 0.0)

    T = N // K
    v = d.reshape(K, T, QB)
    i = (lax.broadcasted_iota(jnp.int32, (K, T, QB), 0) * T
         + lax.broadcasted_iota(jnp.int32, (K, T, QB), 1))
    v, i = _sort_k_dim0(v, i)
    while v.shape[1] > 1:
        v, i = _merge_round(v, i)
    idx_ref[0, 0] = i[:, 0, :] + b * N


def _topk_indices(paug_t, qaug, xxr):
    return pl.pallas_call(
        _topk_body,
        grid=(B, M // QB),
        in_specs=[
            pl.BlockSpec((1, N, 8), lambda b, m: (b, 0, 0)),
            pl.BlockSpec((1, 8, QB), lambda b, m: (b, 0, m)),
            pl.BlockSpec((1, N, 1), lambda b, m: (b, 0, 0)),
        ],
        out_specs=pl.BlockSpec((1, 1, K, QB), lambda b, m: (b, m, 0, 0)),
        out_shape=jax.ShapeDtypeStruct((B, M // QB, K, QB), jnp.int32),
    )(paug_t, qaug, xxr)


_NC, _NS = 2, 16
_NW = _NC * _NS
_TOTAL = B * M * K
_PER_W = _TOTAL // _NW
_GSZ = 128
_NG = _PER_W // _GSZ
_WPB = _NW // B


def _gather_body(feat_hbm, pts_hbm, npts_hbm, idx_hbm, rows_hbm, gp_hbm,
                 idx_v, p_v, q_v, rows_v, gp_v, sem):
    wid = lax.axis_index("s") * _NC + lax.axis_index("c")
    b = wid // _WPB
    pltpu.sync_copy(idx_hbm.at[pl.ds(wid * _PER_W, _PER_W)], idx_v)
    pltpu.sync_copy(pts_hbm.at[pl.ds(b * 3 * N, 3 * N)], p_v)
    pltpu.sync_copy(npts_hbm.at[pl.ds(b * 3 * M, 3 * M)], q_v)
    base = wid * _PER_W
    nbase = b * N
    m_start = (wid % _WPB) * (M // _WPB)
    lane = lax.broadcasted_iota(jnp.int32, (16,), 0)

    def fstep(g, _):
        gslice = idx_v.at[pl.ds(g * _GSZ, _GSZ)]
        pltpu.async_copy(feat_hbm.at[gslice], rows_v, sem).wait()
        pltpu.sync_copy(rows_v, rows_hbm.at[pl.ds(base + g * _GSZ, _GSZ)])
        return 0

    lax.fori_loop(0, _NG, fstep, 0)

    def pstep(i, _):
        ev = idx_v[pl.ds(i * 16, 16)] - nbase
        addr = i * 16 + lane
        midx = m_start + addr // K
        for d in range(3):
            coord = (plsc.load_gather(p_v, [ev + d * N])
                     - plsc.load_gather(q_v, [midx + d * M]))
            plsc.store_scatter(gp_v, [addr + d * _PER_W], coord)
        return 0

    lax.fori_loop(0, _PER_W // 16, pstep, 0)
    for d in range(3):
        pltpu.sync_copy(gp_v.at[pl.ds(d * _PER_W, _PER_W)],
                        gp_hbm.at[pl.ds(d * _TOTAL + base, _PER_W)])


def _sc_gather(feat2d, pts, npts, idx_flat):
    mesh = plsc.VectorSubcoreMesh(core_axis_name="c", subcore_axis_name="s")
    k = functools.partial(
        pl.kernel,
        mesh=mesh,
        out_type=(
            jax.ShapeDtypeStruct((_TOTAL, C), jnp.float32),
            jax.ShapeDtypeStruct((3 * _TOTAL,), jnp.float32),
        ),
        scratch_types=[
            pltpu.VMEM((_PER_W,), jnp.int32),
            pltpu.VMEM((3 * N,), jnp.float32),
            pltpu.VMEM((3 * M,), jnp.float32),
            pltpu.VMEM((_GSZ, C), jnp.float32),
            pltpu.VMEM((3 * _PER_W,), jnp.float32),
            pltpu.SemaphoreType.DMA,
        ],
        compiler_params=pltpu.CompilerParams(needs_layout_passes=False),
    )(_gather_body)
    return k(feat2d, pts, npts, idx_flat)


_RB = 4096


def _asm_body(rows_ref, gp_ref, out_ref):
    out_ref[0, 0:3, :] = gp_ref[...]
    out_ref[0, 3:3 + C, :] = jnp.transpose(rows_ref[...], (1, 0))


def _assemble(rows, gp):
    nb = (M * K) // _RB
    return pl.pallas_call(
        _asm_body,
        grid=(B, nb),
        in_specs=[
            pl.BlockSpec((_RB, C), lambda b, g: (b * nb + g, 0)),
            pl.BlockSpec((3, _RB), lambda b, g: (0, b * nb + g)),
        ],
        out_specs=pl.BlockSpec((1, 3 + C, _RB), lambda b, g: (b, 0, g)),
        out_shape=jax.ShapeDtypeStruct((B, 3 + C, M * K), jnp.float32),
    )(rows, gp)


def kernel(points, new_points, features):
    pts_t = points.transpose(0, 2, 1)
    xxr = jnp.sum(points ** 2, axis=1)[:, :, None]
    zeros5 = jnp.zeros((B, N, 5), jnp.float32)
    paug_t = jnp.concatenate([pts_t, zeros5], axis=-1)

    yy = jnp.sum(new_points ** 2, axis=1)[:, None, :]
    zeros4q = jnp.zeros((B, 4, M), jnp.float32)
    qaug = jnp.concatenate([new_points, yy, zeros4q], axis=1)

    idx = _topk_indices(paug_t, qaug, xxr)
    idx_flat = idx.transpose(0, 1, 3, 2).reshape(_TOTAL)

    feat2d = features.transpose(0, 2, 1).reshape(B * N, C)
    rows, gp = _sc_gather(feat2d, points.reshape(B * 3 * N),
                          new_points.reshape(B * 3 * M), idx_flat)
    out = _assemble(rows, gp.reshape(3, _TOTAL))
    return out.reshape(B, 3 + C, M, K)


# speedup vs baseline: 1.1736x; 1.1736x over previous
import functools

import jax
import jax.numpy as jnp
import numpy as np
from jax import lax
from jax.experimental import pallas as pl
from jax.experimental.pallas import tpu as pltpu
from jax.experimental.pallas import tpu_sc as plsc

B = 4
N = 8192
M = 1024
C = 128
K = 32
QB = 128


def _lex_lt(av, ai, bv, bi):
    return (av < bv) | ((av == bv) & (ai < bi))


def _ce_dim0(v, i, s, L):
    out_v = []
    out_i = []
    for g in range(K // (2 * s)):
        l0 = g * 2 * s
        asc = True if L is None else ((l0 // L) % 2 == 0)
        a_v, a_i = v[l0:l0 + s], i[l0:l0 + s]
        b_v, b_i = v[l0 + s:l0 + 2 * s], i[l0 + s:l0 + 2 * s]
        lt = _lex_lt(a_v, a_i, b_v, b_i)
        if not asc:
            lt = ~lt
        out_v += [jnp.where(lt, a_v, b_v), jnp.where(lt, b_v, a_v)]
        out_i += [jnp.where(lt, a_i, b_i), jnp.where(lt, b_i, a_i)]
    if len(out_v) == 1:
        pass
    v = jnp.concatenate(out_v, 0)
    i = jnp.concatenate(out_i, 0)
    return v, i


def _sort_k_dim0(v, i):
    L = 2
    while L <= K:
        s = L // 2
        while s >= 1:
            v, i = _ce_dim0(v, i, s, None if L == K else L)
            s //= 2
        L *= 2
    return v, i


def _merge_round(v, i):
    t2 = v.shape[1] // 2
    av, ai = v[:, :t2], i[:, :t2]
    rev = list(reversed(range(K)))
    bv = jnp.concatenate([v[j:j + 1, t2:] for j in rev], 0)
    bi = jnp.concatenate([i[j:j + 1, t2:] for j in rev], 0)
    lt = _lex_lt(av, ai, bv, bi)
    nv = jnp.where(lt, av, bv)
    ni = jnp.where(lt, ai, bi)
    s = K // 2
    while s >= 1:
        nv, ni = _ce_dim0(nv, ni, s, None)
        s //= 2
    return nv, ni


def _topk_body(p_ref, q_ref, xx_ref, idx_ref):
    b = pl.program_id(0)
    p = p_ref[0]
    q = q_ref[0]
    xx = xx_ref[0]
    e = lax.dot_general(p, q, (((1,), (0,)), ((), ())),
                        preferred_element_type=jnp.float32)
    d = (xx + q[3:4, :]) - 2.0 * e
    d = jnp.maximum(d, 0.0)

    T = N // K
    v = d.reshape(K, T, QB)
    i = (lax.broadcasted_iota(jnp.int32, (K, T, QB), 0) * T
         + lax.broadcasted_iota(jnp.int32, (K, T, QB), 1))
    v, i = _sort_k_dim0(v, i)
    while v.shape[1] > 1:
        v, i = _merge_round(v, i)
    idx_ref[0, 0] = i[:, 0, :] + b * N


def _topk_indices(paug_t, qaug, xxr):
    return pl.pallas_call(
        _topk_body,
        grid=(B, M // QB),
        in_specs=[
            pl.BlockSpec((1, N, 8), lambda b, m: (b, 0, 0)),
            pl.BlockSpec((1, 8, QB), lambda b, m: (b, 0, m)),
            pl.BlockSpec((1, N, 1), lambda b, m: (b, 0, 0)),
        ],
        out_specs=pl.BlockSpec((1, 1, K, QB), lambda b, m: (b, m, 0, 0)),
        out_shape=jax.ShapeDtypeStruct((B, M // QB, K, QB), jnp.int32),
    )(paug_t, qaug, xxr)


_NC, _NS = 2, 16
_NW = _NC * _NS
_TOTAL = B * M * K
_PER_W = _TOTAL // _NW
_GSZ = 128
_NG = _PER_W // _GSZ
_WPB = _NW // B


def _gather_body(feat_hbm, pts_hbm, npts_hbm, idx_hbm, rows_hbm, gp_hbm,
                 idx_v, p_v, q_v, rows_v, gp_v, sem):
    wid = lax.axis_index("s") * _NC + lax.axis_index("c")
    b = wid // _WPB
    pltpu.sync_copy(idx_hbm.at[pl.ds(wid * _PER_W, _PER_W)], idx_v)
    pltpu.sync_copy(pts_hbm.at[pl.ds(b * 3 * N, 3 * N)], p_v)
    pltpu.sync_copy(npts_hbm.at[pl.ds(b * 3 * M, 3 * M)], q_v)
    base = wid * _PER_W
    nbase = b * N
    m_start = (wid % _WPB) * (M // _WPB)
    lane = lax.broadcasted_iota(jnp.int32, (16,), 0)

    def fstep(g, _):
        gslice = idx_v.at[pl.ds(g * _GSZ, _GSZ)]
        pltpu.async_copy(feat_hbm.at[gslice], rows_v, sem).wait()
        pltpu.sync_copy(rows_v, rows_hbm.at[pl.ds(base + g * _GSZ, _GSZ)])
        return 0

    lax.fori_loop(0, _NG, fstep, 0)

    def pstep(i, _):
        ev = idx_v[pl.ds(i * 16, 16)] - nbase
        addr = i * 16 + lane
        midx = m_start + addr // K
        for d in range(3):
            coord = (plsc.load_gather(p_v, [ev + d * N])
                     - plsc.load_gather(q_v, [midx + d * M]))
            plsc.store_scatter(gp_v, [addr + d * _PER_W], coord)
        return 0

    lax.fori_loop(0, _PER_W // 16, pstep, 0)
    for d in range(3):
        pltpu.sync_copy(gp_v.at[pl.ds(d * _PER_W, _PER_W)],
                        gp_hbm.at[pl.ds(d * _TOTAL + base, _PER_W)])


def _sc_gather(feat2d, pts, npts, idx_flat):
    mesh = plsc.VectorSubcoreMesh(core_axis_name="c", subcore_axis_name="s")
    k = functools.partial(
        pl.kernel,
        mesh=mesh,
        out_type=(
            jax.ShapeDtypeStruct((_TOTAL, C), jnp.float32),
            jax.ShapeDtypeStruct((3 * _TOTAL,), jnp.float32),
        ),
        scratch_types=[
            pltpu.VMEM((_PER_W,), jnp.int32),
            pltpu.VMEM((3 * N,), jnp.float32),
            pltpu.VMEM((3 * M,), jnp.float32),
            pltpu.VMEM((_GSZ, C), jnp.float32),
            pltpu.VMEM((3 * _PER_W,), jnp.float32),
            pltpu.SemaphoreType.DMA,
        ],
        compiler_params=pltpu.CompilerParams(needs_layout_passes=False),
    )(_gather_body)
    return k(feat2d, pts, npts, idx_flat)


_RB = 4096


def _asm_body(rows_ref, gp_ref, out_ref):
    out_ref[0, 0:3, :] = gp_ref[...]
    out_ref[0, 3:3 + C, :] = jnp.transpose(rows_ref[...], (1, 0))


def _assemble(rows, gp):
    nb = (M * K) // _RB
    return pl.pallas_call(
        _asm_body,
        grid=(B, nb),
        in_specs=[
            pl.BlockSpec((_RB, C), lambda b, g: (b * nb + g, 0)),
            pl.BlockSpec((3, _RB), lambda b, g: (0, b * nb + g)),
        ],
        out_specs=pl.BlockSpec((1, 3 + C, _RB), lambda b, g: (b, 0, g)),
        out_shape=jax.ShapeDtypeStruct((B, 3 + C, M * K), jnp.float32),
    )(rows, gp)


def kernel(points, new_points, features):
    pts_t = points.transpose(0, 2, 1)
    xxr = jnp.sum(points ** 2, axis=1)[:, :, None]
    zeros5 = jnp.zeros((B, N, 5), jnp.float32)
    paug_t = jnp.concatenate([pts_t, zeros5], axis=-1)

    yy = jnp.sum(new_points ** 2, axis=1)[:, None, :]
    zeros4q = jnp.zeros((B, 4, M), jnp.float32)
    qaug = jnp.concatenate([new_points, yy, zeros4q], axis=1)

    idx = _topk_indices(paug_t, qaug, xxr)
    idx_flat = idx.transpose(0, 1, 3, 2).reshape(_TOTAL)

    feat2d = features.transpose(0, 2, 1).reshape(B * N, C)
    rows, gp = _sc_gather(feat2d, points.reshape(B * 3 * N),
                          new_points.reshape(B * 3 * M), idx_flat)
    gf = rows.reshape(B, M, K, C).transpose(0, 3, 1, 2)
    gpt = gp.reshape(3, B, M, K).transpose(1, 0, 2, 3)
    return jnp.concatenate([gpt, gf], axis=1)
